# baseline (speedup 1.0000x reference)
import functools
import math

import jax
import jax.numpy as jnp
from jax import lax
from jax.experimental import pallas as pl
from jax.experimental.pallas import tpu as pltpu
from jax.experimental.pallas import tpu_sc as plsc

_NC = 2
_NS = 16
_NW = _NC * _NS

_CHUNK = 64


@functools.lru_cache(maxsize=None)
def _make_sc_gather(vocab, emb, n_idx, tok0, n_tok):
    assert n_tok % (_NW * _CHUNK) == 0 and tok0 % 8 == 0
    n_per_w = n_tok // _NW
    n_ch = n_per_w // _CHUNK
    assert n_per_w * emb * 4 <= 480 * 1024
    mesh = plsc.VectorSubcoreMesh(core_axis_name="c", subcore_axis_name="s")

    @functools.partial(
        pl.kernel,
        mesh=mesh,
        out_type=jax.ShapeDtypeStruct((n_tok, emb), jnp.float32),
        compiler_params=pltpu.CompilerParams(use_tc_tiling_on_sc=True),
        scratch_types=[
            pltpu.VMEM((n_per_w,), jnp.int32),
            pltpu.VMEM((n_per_w, emb), jnp.float32),
            pltpu.SemaphoreType.DMA,
            pltpu.SemaphoreType.DMA,
            pltpu.SemaphoreType.DMA,
        ],
    )
    def gather_kernel(table_hbm, idx_hbm, out_hbm, idx_v, rows_v, sa, sb, sw):
        wid = lax.axis_index("s") * _NC + lax.axis_index("c")
        base = wid * n_per_w
        pltpu.sync_copy(idx_hbm.at[pl.ds(tok0 + base, n_per_w)], idx_v)
        half = (n_ch // 2) * _CHUNK
        waves = []
        for lo, hi, sem in ((0, half, sa), (half, n_per_w, sb)):
            cps = []
            for off in range(lo, hi, _CHUNK):
                cps.append(
                    pltpu.async_copy(
                        table_hbm.at[idx_v.at[pl.ds(off, _CHUNK)]],
                        rows_v.at[pl.ds(off, _CHUNK)],
                        sem,
                    )
                )
            waves.append(cps)
        writes = []
        for (lo, hi, _), cps in zip(
            ((0, half, sa), (half, n_per_w, sb)), waves
        ):
            for cp in cps:
                cp.wait()
            if hi > lo:
                writes.append(
                    pltpu.async_copy(
                        rows_v.at[pl.ds(lo, hi - lo)],
                        out_hbm.at[pl.ds(base + lo, hi - lo)],
                        sw,
                    )
                )
        for wcp in writes:
            wcp.wait()

    return gather_kernel


_PB = 2


@functools.lru_cache(maxsize=None)
def _make_tc_matmul(batch, seq, emb, hid, scale, l0, lc, aliased):
    pb = _PB if lc % _PB == 0 and l0 % _PB == 0 else 1
    grid = (lc // pb,)

    def compute(g_ref, pe_ref, w_ref, b_ref, o_ref):
        w = w_ref[...]
        bias = b_ref[...]
        g = g_ref[...] * scale
        for j in range(pb):
            a = (
                lax.slice(g, (j * batch, 0), ((j + 1) * batch, emb))
                + pe_ref[j]
            )
            o_ref[j] = (
                jnp.dot(a, w, preferred_element_type=jnp.float32) + bias
            )

    if aliased:

        def body(g_ref, pe_ref, w_ref, b_ref, _prev_ref, o_ref):
            compute(g_ref, pe_ref, w_ref, b_ref, o_ref)

    else:
        body = compute

    in_specs = [
        pl.BlockSpec((pb * batch, emb), lambda i: (i, 0)),
        pl.BlockSpec((pb, 1, emb), lambda i: (i + l0 // pb, 0, 0)),
        pl.BlockSpec((emb, hid), lambda i: (0, 0)),
        pl.BlockSpec((1, hid), lambda i: (0, 0)),
    ]
    if aliased:
        in_specs.append(pl.BlockSpec(memory_space=pl.ANY))
    return pl.pallas_call(
        body,
        grid=grid,
        in_specs=in_specs,
        out_specs=pl.BlockSpec((pb, batch, hid), lambda i: (i + l0 // pb, 0, 0)),
        out_shape=jax.ShapeDtypeStruct((seq, batch, hid), jnp.float32),
        input_output_aliases={4: 0} if aliased else {},
    )


_CHUNK_POS = (2, 8, 16, 24)


def kernel(x, tok_emb, fc_w, fc_b, pe):
    b, l = x.shape
    vocab, emb = tok_emb.shape
    hid = fc_w.shape[1]
    n_tok = b * l
    scale = math.sqrt(emb)

    idx = x.T.reshape(n_tok)
    pe_t = pe[:, :l, :].transpose(1, 0, 2)
    bias = fc_b.reshape(1, hid)

    chunks = (
        _CHUNK_POS
        if sum(_CHUNK_POS) == l and all(c * b % (_NW * _CHUNK) == 0 for c in _CHUNK_POS)
        else (l,)
    )
    out_t = None
    l0 = 0
    for c, lc in enumerate(chunks):
        nc = lc * b
        g_c = _make_sc_gather(vocab, emb, n_tok, l0 * b, nc)(tok_emb, idx)
        mm = _make_tc_matmul(b, l, emb, hid, scale, l0, lc, c > 0)
        if c == 0:
            out_t = mm(g_c, pe_t, fc_w, bias)
        else:
            out_t = mm(g_c, pe_t, fc_w, bias, out_t)
        l0 += lc
    return out_t.transpose(1, 0, 2)


# speedup vs baseline: 1.0042x; 1.0042x over previous
import functools
import math

import jax
import jax.numpy as jnp
from jax import lax
from jax.experimental import pallas as pl
from jax.experimental.pallas import tpu as pltpu
from jax.experimental.pallas import tpu_sc as plsc

_NC = 2
_NS = 16
_NW = _NC * _NS

_CHUNK = 64


@functools.lru_cache(maxsize=None)
def _make_sc_gather(vocab, emb, n_idx, tok0, n_tok):
    n_per_w = n_tok // _NW
    ch = 128 if n_per_w % 128 == 0 else _CHUNK
    assert n_tok % (_NW * ch) == 0 and tok0 % 8 == 0
    n_ch = n_per_w // ch
    assert n_per_w * emb * 4 <= 480 * 1024
    mesh = plsc.VectorSubcoreMesh(core_axis_name="c", subcore_axis_name="s")

    @functools.partial(
        pl.kernel,
        mesh=mesh,
        out_type=jax.ShapeDtypeStruct((n_tok, emb), jnp.float32),
        compiler_params=pltpu.CompilerParams(use_tc_tiling_on_sc=True),
        scratch_types=[
            pltpu.VMEM((n_per_w,), jnp.int32),
            pltpu.VMEM((n_per_w, emb), jnp.float32),
            pltpu.SemaphoreType.DMA,
            pltpu.SemaphoreType.DMA,
            pltpu.SemaphoreType.DMA,
        ],
    )
    def gather_kernel(table_hbm, idx_hbm, out_hbm, idx_v, rows_v, sa, sb, sw):
        wid = lax.axis_index("s") * _NC + lax.axis_index("c")
        base = wid * n_per_w
        pltpu.sync_copy(idx_hbm.at[pl.ds(tok0 + base, n_per_w)], idx_v)
        half = (n_ch // 2) * ch
        waves = []
        for lo, hi, sem in ((0, half, sa), (half, n_per_w, sb)):
            cps = []
            for off in range(lo, hi, ch):
                cps.append(
                    pltpu.async_copy(
                        table_hbm.at[idx_v.at[pl.ds(off, ch)]],
                        rows_v.at[pl.ds(off, ch)],
                        sem,
                    )
                )
            waves.append(cps)
        writes = []
        for (lo, hi, _), cps in zip(
            ((0, half, sa), (half, n_per_w, sb)), waves
        ):
            for cp in cps:
                cp.wait()
            if hi > lo:
                writes.append(
                    pltpu.async_copy(
                        rows_v.at[pl.ds(lo, hi - lo)],
                        out_hbm.at[pl.ds(base + lo, hi - lo)],
                        sw,
                    )
                )
        for wcp in writes:
            wcp.wait()

    return gather_kernel


_PB = 2


@functools.lru_cache(maxsize=None)
def _make_tc_matmul(batch, seq, emb, hid, scale, l0, lc, aliased):
    pb = _PB if lc % _PB == 0 and l0 % _PB == 0 else 1
    grid = (lc // pb,)

    def compute(g_ref, pe_ref, w_ref, b_ref, o_ref):
        w = w_ref[...]
        bias = b_ref[...]
        g = g_ref[...] * scale
        for j in range(pb):
            a = (
                lax.slice(g, (j * batch, 0), ((j + 1) * batch, emb))
                + pe_ref[j]
            )
            o_ref[j] = (
                jnp.dot(a, w, preferred_element_type=jnp.float32) + bias
            )

    if aliased:

        def body(g_ref, pe_ref, w_ref, b_ref, _prev_ref, o_ref):
            compute(g_ref, pe_ref, w_ref, b_ref, o_ref)

    else:
        body = compute

    in_specs = [
        pl.BlockSpec((pb * batch, emb), lambda i: (i, 0)),
        pl.BlockSpec((pb, 1, emb), lambda i: (i + l0 // pb, 0, 0)),
        pl.BlockSpec((emb, hid), lambda i: (0, 0)),
        pl.BlockSpec((1, hid), lambda i: (0, 0)),
    ]
    if aliased:
        in_specs.append(pl.BlockSpec(memory_space=pl.ANY))
    return pl.pallas_call(
        body,
        grid=grid,
        in_specs=in_specs,
        out_specs=pl.BlockSpec((pb, batch, hid), lambda i: (i + l0 // pb, 0, 0)),
        out_shape=jax.ShapeDtypeStruct((seq, batch, hid), jnp.float32),
        input_output_aliases={4: 0} if aliased else {},
    )


_CHUNK_POS = (2, 8, 16, 24)


def kernel(x, tok_emb, fc_w, fc_b, pe):
    b, l = x.shape
    vocab, emb = tok_emb.shape
    hid = fc_w.shape[1]
    n_tok = b * l
    scale = math.sqrt(emb)

    idx = x.T.reshape(n_tok)
    pe_t = pe[:, :l, :].transpose(1, 0, 2)
    bias = fc_b.reshape(1, hid)

    chunks = (
        _CHUNK_POS
        if sum(_CHUNK_POS) == l and all(c * b % (_NW * _CHUNK) == 0 for c in _CHUNK_POS)
        else (l,)
    )
    out_t = None
    l0 = 0
    for c, lc in enumerate(chunks):
        nc = lc * b
        g_c = _make_sc_gather(vocab, emb, n_tok, l0 * b, nc)(tok_emb, idx)
        mm = _make_tc_matmul(b, l, emb, hid, scale, l0, lc, c > 0)
        if c == 0:
            out_t = mm(g_c, pe_t, fc_w, bias)
        else:
            out_t = mm(g_c, pe_t, fc_w, bias, out_t)
        l0 += lc
    return out_t.transpose(1, 0, 2)
